# baseline (speedup 1.0000x reference)
import functools
import math

import jax
import jax.numpy as jnp
import numpy as np
from jax import lax
from jax.experimental import pallas as pl
from jax.experimental.pallas import tpu as pltpu
from jax.experimental.pallas import tpu_sc as plsc

N_NODES = 10000
N_EDGES = 320000
FEAT = 128
N_RBF = 20
CUTOFF = 5.0

PAD = 16
E_PAD = 327680
EDGE_BLK = 8192
FOLD = 8
FB = EDGE_BLK // FOLD
N_BLOCKS = E_PAD // EDGE_BLK
COLS = E_PAD // FOLD

NC = 2
NS = 16
NW = NC * NS
E_PER_W = E_PAD // NW
CHUNK = 64
N_CHUNKS = E_PER_W // CHUNK
GRP = 5
N_ACC = 10016
TRASH = N_NODES
ACC_STRIPE = N_ACC // NS

NODE_BLK = 10000
NODE_RPB = NODE_BLK // 8


def _node_mlp_body(s_ref, w1t_ref, b1_ref, wp_ref, b2p_ref, out_ref):
    x = jnp.dot(s_ref[...], w1t_ref[...], preferred_element_type=jnp.float32)
    x = x + b1_ref[...]
    h = x * jax.nn.sigmoid(x)
    ph = jnp.dot(h, wp_ref[...], preferred_element_type=jnp.float32) + b2p_ref[...]
    out_ref[...] = jnp.concatenate(
        [ph[c * NODE_RPB : (c + 1) * NODE_RPB, :] for c in range(8)], axis=1
    )


def _node_mlp(s_perm, w1t, b1r, wp, b2p):
    return pl.pallas_call(
        _node_mlp_body,
        grid=(1,),
        in_specs=[
            pl.BlockSpec((NODE_BLK, FEAT), lambda i: (0, 0)),
            pl.BlockSpec((FEAT, FEAT), lambda i: (0, 0)),
            pl.BlockSpec((1, FEAT), lambda i: (0, 0)),
            pl.BlockSpec((FEAT, PAD), lambda i: (0, 0)),
            pl.BlockSpec((1, PAD), lambda i: (0, 0)),
        ],
        out_specs=pl.BlockSpec((NODE_RPB, 8 * PAD), lambda i: (0, 0)),
        out_shape=jax.ShapeDtypeStruct((N_NODES // 8, 8 * PAD), jnp.float32),
    )(s_perm, w1t, b1r, wp, b2p)


def _chunk_coords(wid, ch):
    e0 = wid * E_PER_W + ch * CHUNK
    blk = e0 // EDGE_BLK
    t0 = e0 - blk * EDGE_BLK
    a = t0 // FB
    col0 = blk * FB + (t0 - a * FB)
    return a, col0


def _gather_body(table_hbm, idx_hbm, out_hbm, idx_v, rows_v, soa_v, gsem, ssem):
    c = lax.axis_index("c")
    s = lax.axis_index("s")
    wid = s * NC + c
    pltpu.sync_copy(idx_hbm.at[wid], idx_v)

    def grp(g, carry):
        base = g * GRP
        cps = [
            pltpu.async_copy(table_hbm.at[idx_v.at[base + j]], rows_v.at[j], gsem)
            for j in range(GRP)
        ]
        for cp in cps:
            cp.wait()

        iota16 = lax.broadcasted_iota(jnp.int32, (16,), 0)
        for j2 in range(GRP):
            bufv = jnp.full((16,), j2, jnp.int32)
            for jj in range(PAD):
                colv = jnp.full((16,), jj, jnp.int32)
                for g5 in range(CHUNK // 16):
                    vals = plsc.load_gather(rows_v, [bufv, iota16 + 16 * g5, colv])
                    soa_v[j2, jj, pl.ds(16 * g5, 16)] = vals

        sts = []
        for j2 in range(GRP):
            a, col0 = _chunk_coords(wid, base + j2)
            sts.append(
                pltpu.async_copy(
                    soa_v.at[j2], out_hbm.at[:, a, pl.ds(col0, CHUNK)], ssem
                )
            )
        for st in sts:
            st.wait()
        return carry

    lax.fori_loop(0, N_CHUNKS // GRP, grp, 0)


def _sc_gather(node16, src3):
    mesh = plsc.VectorSubcoreMesh(core_axis_name="c", subcore_axis_name="s")
    f = pl.kernel(
        _gather_body,
        out_type=jax.ShapeDtypeStruct((PAD, FOLD, COLS), jnp.float32),
        mesh=mesh,
        compiler_params=pltpu.CompilerParams(use_tc_tiling_on_sc=False, needs_layout_passes=False),
        scratch_types=[
            pltpu.VMEM((N_CHUNKS, CHUNK), jnp.int32),
            pltpu.VMEM((GRP, CHUNK, PAD), jnp.float32),
            pltpu.VMEM((GRP, PAD, CHUNK), jnp.float32),
            pltpu.SemaphoreType.DMA,
            pltpu.SemaphoreType.DMA,
        ],
    )
    return f(node16, src3)


def _edge_body(p_ref, r_ref, v_ref, wdk_ref, bdk_ref, ck_ref, ak_ref, out_ref):
    rs = r_ref[...]
    xx = rs[0:8]
    yy = rs[8:16]
    zz = rs[16:24]
    d2 = xx * xx + yy * yy + zz * zz
    d = jnp.sqrt(d2)
    d3 = jnp.concatenate([d, d, d], axis=0)
    unit = rs / d3
    idm = jnp.where(d2 == 0.0, 0.0, 1.0 / d)
    x1 = d * (math.pi / CUTOFF)
    s1 = jnp.sin(x1)
    c1 = jnp.cos(x1)
    two_c1 = c1 + c1
    terms = [s1, two_c1 * s1]
    for _ in range(N_RBF - 2):
        terms.append(two_c1 * terms[-1] - terms[-2])
    rbf = jnp.concatenate([t * idm for t in terms], axis=0)
    w = jnp.dot(wdk_ref[...], rbf, preferred_element_type=jnp.float32) + bdk_ref[...]
    P = p_ref[...] * w
    u = jnp.concatenate(
        [unit[8 * k : 8 * (k + 1)] for _i in range(3) for k in range(3)]
        + [jnp.ones((24, FB), jnp.float32), jnp.zeros((32, FB), jnp.float32)],
        axis=0,
    )
    vpad = jnp.concatenate(
        [v_ref[...], jnp.zeros((128 - 72, FB), jnp.float32)], axis=0
    )
    out_ref[...] = (
        jnp.dot(ck_ref[...], P, preferred_element_type=jnp.float32) * u
        + jnp.dot(ak_ref[...], P, preferred_element_type=jnp.float32) * vpad
    )


def _edge_stage(p, r_stack, v_stack, wdk, bdk, ck, ak):
    return pl.pallas_call(
        _edge_body,
        grid=(N_BLOCKS,),
        in_specs=[
            pl.BlockSpec((128, FB), lambda i: (0, i)),
            pl.BlockSpec((24, FB), lambda i: (0, i)),
            pl.BlockSpec((72, FB), lambda i: (0, i)),
            pl.BlockSpec((128, 8 * N_RBF), lambda i: (0, 0)),
            pl.BlockSpec((128, 1), lambda i: (0, 0)),
            pl.BlockSpec((128, 128), lambda i: (0, 0)),
            pl.BlockSpec((128, 128), lambda i: (0, 0)),
        ],
        out_specs=pl.BlockSpec((128, FB), lambda i: (0, i)),
        out_shape=jax.ShapeDtypeStruct((128, COLS), jnp.float32),
    )(p, r_stack, v_stack, wdk, bdk, ck, ak)


def _scatter_body(vals_hbm, dst_hbm, zeros_hbm, out_hbm, idx_v, soa_v, aos_v,
                  acc, lsem, ssem):
    c = lax.axis_index("c")
    s = lax.axis_index("s")
    wid = s * NC + c
    pltpu.sync_copy(
        zeros_hbm.at[pl.ds(s * ACC_STRIPE, ACC_STRIPE)],
        acc.at[pl.ds(s * ACC_STRIPE, ACC_STRIPE)],
    )
    plsc.subcore_barrier()
    pltpu.sync_copy(dst_hbm.at[wid], idx_v)

    def grp(g, carry):
        base = g * GRP
        lds = []
        for j2 in range(GRP):
            a, col0 = _chunk_coords(wid, base + j2)
            lds.append(
                pltpu.async_copy(
                    vals_hbm.at[:, a, pl.ds(col0, CHUNK)], soa_v.at[j2], lsem
                )
            )
        for ld in lds:
            ld.wait()

        iota16 = lax.broadcasted_iota(jnp.int32, (16,), 0)
        for j2 in range(GRP):
            bufv = jnp.full((16,), j2, jnp.int32)
            for jj in range(PAD):
                colv = jnp.full((16,), jj, jnp.int32)
                for g5 in range(CHUNK // 16):
                    vals = soa_v[j2, jj, pl.ds(16 * g5, 16)]
                    plsc.store_scatter(aos_v, [bufv, iota16 + 16 * g5, colv], vals)

        scs = [
            pltpu.async_copy(
                aos_v.at[j2], acc.at[idx_v.at[base + j2]], ssem, add=True
            )
            for j2 in range(GRP)
        ]
        for sc in scs:
            sc.wait()
        return carry

    lax.fori_loop(0, N_CHUNKS // GRP, grp, 0)
    plsc.subcore_barrier()
    pltpu.sync_copy(
        acc.at[pl.ds(s * ACC_STRIPE, ACC_STRIPE)],
        out_hbm.at[c, pl.ds(s * ACC_STRIPE, ACC_STRIPE)],
    )


def _sc_scatter(vals3, dst3, zeros):
    mesh = plsc.VectorSubcoreMesh(core_axis_name="c", subcore_axis_name="s")
    f = pl.kernel(
        _scatter_body,
        out_type=jax.ShapeDtypeStruct((NC, N_ACC, PAD), jnp.float32),
        mesh=mesh,
        compiler_params=pltpu.CompilerParams(use_tc_tiling_on_sc=False, needs_layout_passes=False),
        scratch_types=[
            pltpu.VMEM((N_CHUNKS, CHUNK), jnp.int32),
            pltpu.VMEM((GRP, PAD, CHUNK), jnp.float32),
            pltpu.VMEM((GRP, CHUNK, PAD), jnp.float32),
            pltpu.VMEM_SHARED((N_ACC, PAD), jnp.float32),
            pltpu.SemaphoreType.DMA,
            pltpu.SemaphoreType.DMA,
        ],
    )
    return f(vals3, dst3, zeros)


def _combine_body(p_ref, s_out, v_out):
    tot = p_ref[0] + p_ref[1]
    s_out[...] = tot[0:N_NODES, 9:12]
    v_out[...] = tot[0:N_NODES, 0:9]


def _combine(partials):
    return pl.pallas_call(
        _combine_body,
        in_specs=[pl.BlockSpec((NC, N_ACC, PAD), lambda: (0, 0, 0))],
        out_specs=[
            pl.BlockSpec((N_NODES, 3), lambda: (0, 0)),
            pl.BlockSpec((N_NODES, 9), lambda: (0, 0)),
        ],
        out_shape=[
            jax.ShapeDtypeStruct((N_NODES, 3), jnp.float32),
            jax.ShapeDtypeStruct((N_NODES, 9), jnp.float32),
        ],
    )(partials)


def kernel(s_j, v_j, r_ij, nbrs, W1, b1, W2, b2, Wd, bd):
    w1t = W1.T
    b1r = b1.reshape(1, FEAT)
    wp = jnp.zeros((FEAT, PAD), jnp.float32).at[:, :9].set(W2[:9].T)
    b2p = jnp.zeros((1, PAD), jnp.float32).at[0, :9].set(b2[:9])
    wd16 = jnp.zeros((PAD, N_RBF), jnp.float32).at[:9].set(Wd[:9])
    bd16 = jnp.zeros((PAD,), jnp.float32).at[:9].set(bd[:9])
    wdk = jnp.kron(wd16, jnp.eye(FOLD, dtype=jnp.float32))
    bdk = jnp.repeat(bd16, FOLD).reshape(128, 1)

    c16 = np.zeros((PAD, PAD), np.float32)
    a16 = np.zeros((PAD, PAD), np.float32)
    for i in range(3):
        for k in range(3):
            c16[3 * i + k, 6 + i] = 1.0
            a16[3 * i + k, i] = 1.0
        c16[9 + i, 3 + i] = 1.0
    ck = jnp.asarray(np.kron(c16, np.eye(8, dtype=np.float32)))
    ak = jnp.asarray(np.kron(a16, np.eye(8, dtype=np.float32)))

    q = np.arange(N_NODES)
    qi, qt = q // NODE_BLK, q % NODE_BLK
    n_of_q = jnp.asarray((qi * NODE_RPB + (qt % NODE_RPB)) * 8 + qt // NODE_RPB)
    s_perm = jnp.take(s_j, n_of_q, axis=0)

    npad = E_PAD - N_EDGES
    src3 = jnp.concatenate(
        [nbrs[:, 1], jnp.zeros((npad,), jnp.int32)]
    ).reshape(NW, N_CHUNKS, CHUNK)
    dst3 = jnp.concatenate(
        [nbrs[:, 0], jnp.full((npad,), TRASH, jnp.int32)]
    ).reshape(NW, N_CHUNKS, CHUNK)

    r_pad = jnp.concatenate([r_ij, jnp.zeros((npad, 3), jnp.float32)], axis=0)
    v_pad = jnp.concatenate(
        [v_j.reshape(N_EDGES, 9), jnp.zeros((npad, 9), jnp.float32)], axis=0
    )
    r_stack = (
        r_pad.T.reshape(3, N_BLOCKS, FOLD, FB).transpose(0, 2, 1, 3).reshape(24, COLS)
    )
    v_stack = (
        v_pad.T.reshape(9, N_BLOCKS, FOLD, FB).transpose(0, 2, 1, 3).reshape(72, COLS)
    )
    zeros = jnp.zeros((N_ACC, PAD), jnp.float32)

    node_tbl = _node_mlp(s_perm, w1t, b1r, wp, b2p).reshape(N_NODES, PAD)
    p3 = _sc_gather(node_tbl, src3)
    vals = _edge_stage(p3.reshape(128, COLS), r_stack, v_stack, wdk, bdk, ck, ak)
    partials = _sc_scatter(vals.reshape(PAD, FOLD, COLS), dst3, zeros)
    ds, dv = _combine(partials)
    return (ds, dv.reshape(N_NODES, 3, 3))


# speedup vs baseline: 1.2857x; 1.2857x over previous
import functools
import math

import jax
import jax.numpy as jnp
import numpy as np
from jax import lax
from jax.experimental import pallas as pl
from jax.experimental.pallas import tpu as pltpu
from jax.experimental.pallas import tpu_sc as plsc

N_NODES = 10000
N_EDGES = 320000
FEAT = 128
N_RBF = 20
CUTOFF = 5.0

PAD = 16
E_PAD = 327680
EDGE_BLK = 8192
FOLD = 8
FB = EDGE_BLK // FOLD
N_BLOCKS = E_PAD // EDGE_BLK
COLS = E_PAD // FOLD

NC = 2
NS = 16
NW = NC * NS
E_PER_W = E_PAD // NW
CHUNK = 80
N_CHUNKS = E_PER_W // CHUNK
GRP = 4
N_ACC = 10016
TRASH = N_NODES
ACC_STRIPE = N_ACC // NS

NODE_BLK = 10000
NODE_RPB = NODE_BLK // 8


def _node_mlp_body(s_ref, w1t_ref, b1_ref, wp_ref, b2p_ref, out_ref):
    x = jnp.dot(s_ref[...], w1t_ref[...], preferred_element_type=jnp.float32)
    x = x + b1_ref[...]
    h = x * jax.nn.sigmoid(x)
    ph = jnp.dot(h, wp_ref[...], preferred_element_type=jnp.float32) + b2p_ref[...]
    out_ref[...] = jnp.concatenate(
        [ph[c * NODE_RPB : (c + 1) * NODE_RPB, :] for c in range(8)], axis=1
    )


def _node_mlp(s_perm, w1t, b1r, wp, b2p):
    return pl.pallas_call(
        _node_mlp_body,
        grid=(1,),
        in_specs=[
            pl.BlockSpec((NODE_BLK, FEAT), lambda i: (0, 0)),
            pl.BlockSpec((FEAT, FEAT), lambda i: (0, 0)),
            pl.BlockSpec((1, FEAT), lambda i: (0, 0)),
            pl.BlockSpec((FEAT, PAD), lambda i: (0, 0)),
            pl.BlockSpec((1, PAD), lambda i: (0, 0)),
        ],
        out_specs=pl.BlockSpec((NODE_RPB, 8 * PAD), lambda i: (0, 0)),
        out_shape=jax.ShapeDtypeStruct((N_NODES // 8, 8 * PAD), jnp.float32),
    )(s_perm, w1t, b1r, wp, b2p)


def _gather_body(table_hbm, idx_hbm, out_hbm, idx_v, rows_v, gsem, ssem):
    c = lax.axis_index("c")
    s = lax.axis_index("s")
    wid = s * NC + c
    pltpu.sync_copy(idx_hbm.at[wid], idx_v)

    def grp(g, carry):
        base = g * GRP
        cps = [
            pltpu.async_copy(table_hbm.at[idx_v.at[base + j]], rows_v.at[j], gsem)
            for j in range(GRP)
        ]
        for cp in cps:
            cp.wait()
        sts = [
            pltpu.async_copy(rows_v.at[j], out_hbm.at[wid, base + j], ssem)
            for j in range(GRP)
        ]
        for st in sts:
            st.wait()
        return carry

    lax.fori_loop(0, N_CHUNKS // GRP, grp, 0)


def _sc_gather(node16, src3):
    mesh = plsc.VectorSubcoreMesh(core_axis_name="c", subcore_axis_name="s")
    f = pl.kernel(
        _gather_body,
        out_type=jax.ShapeDtypeStruct((NW, N_CHUNKS, CHUNK, PAD), jnp.float32),
        mesh=mesh,
        compiler_params=pltpu.CompilerParams(use_tc_tiling_on_sc=False, needs_layout_passes=False),
        scratch_types=[
            pltpu.VMEM((N_CHUNKS, CHUNK), jnp.int32),
            pltpu.VMEM((GRP, CHUNK, PAD), jnp.float32),
            pltpu.SemaphoreType.DMA,
            pltpu.SemaphoreType.DMA,
        ],
    )
    return f(node16, src3)


def _edge_body(p_ref, r_ref, v_ref, wdk_ref, bdk_ref, ck_ref, ak_ref,
               pin_ref, pout_ref, out_ref):
    rs = r_ref[...]
    xx = rs[0:8]
    yy = rs[8:16]
    zz = rs[16:24]
    d2 = xx * xx + yy * yy + zz * zz
    d = jnp.sqrt(d2)
    d3 = jnp.concatenate([d, d, d], axis=0)
    d23 = jnp.concatenate([d2, d2, d2], axis=0)
    unit = jnp.where(d23 == 0.0, 0.0, rs / d3)
    idm = jnp.where(d2 == 0.0, 0.0, 1.0 / d)
    x1 = d * (math.pi / CUTOFF)
    s1 = jnp.sin(x1)
    c1 = jnp.cos(x1)
    two_c1 = c1 + c1
    terms = [s1, two_c1 * s1]
    for _ in range(N_RBF - 2):
        terms.append(two_c1 * terms[-1] - terms[-2])
    rbf = jnp.concatenate([t * idm for t in terms], axis=0)
    w = jnp.dot(wdk_ref[...], rbf, preferred_element_type=jnp.float32) + bdk_ref[...]
    p_stack = jnp.dot(
        pin_ref[...], p_ref[...].T, preferred_element_type=jnp.float32
    )
    P = p_stack * w
    u = jnp.concatenate(
        [unit[8 * k : 8 * (k + 1)] for _i in range(3) for k in range(3)]
        + [jnp.ones((24, FB), jnp.float32), jnp.zeros((32, FB), jnp.float32)],
        axis=0,
    )
    vpad = jnp.concatenate(
        [v_ref[...], jnp.zeros((128 - 72, FB), jnp.float32)], axis=0
    )
    outt = (
        jnp.dot(ck_ref[...], P, preferred_element_type=jnp.float32) * u
        + jnp.dot(ak_ref[...], P, preferred_element_type=jnp.float32) * vpad
    )
    out_ref[...] = jnp.dot(
        pout_ref[...], outt, preferred_element_type=jnp.float32
    ).T


def _edge_stage(p, r_stack, v_stack, wdk, bdk, ck, ak, pin, pout):
    return pl.pallas_call(
        _edge_body,
        grid=(N_BLOCKS,),
        in_specs=[
            pl.BlockSpec((FB, 128), lambda i: (i, 0)),
            pl.BlockSpec((24, FB), lambda i: (0, i)),
            pl.BlockSpec((72, FB), lambda i: (0, i)),
            pl.BlockSpec((128, 8 * N_RBF), lambda i: (0, 0)),
            pl.BlockSpec((128, 1), lambda i: (0, 0)),
            pl.BlockSpec((128, 128), lambda i: (0, 0)),
            pl.BlockSpec((128, 128), lambda i: (0, 0)),
            pl.BlockSpec((128, 128), lambda i: (0, 0)),
            pl.BlockSpec((128, 128), lambda i: (0, 0)),
        ],
        out_specs=pl.BlockSpec((FB, 128), lambda i: (i, 0)),
        out_shape=jax.ShapeDtypeStruct((E_PAD // 8, 128), jnp.float32),
    )(p, r_stack, v_stack, wdk, bdk, ck, ak, pin, pout)


def _scatter_body(vals_hbm, dst_hbm, zeros_hbm, out_hbm, idx_v, vals_v,
                  acc, lsem, ssem):
    c = lax.axis_index("c")
    s = lax.axis_index("s")
    wid = s * NC + c
    pltpu.sync_copy(
        zeros_hbm.at[pl.ds(s * ACC_STRIPE, ACC_STRIPE)],
        acc.at[pl.ds(s * ACC_STRIPE, ACC_STRIPE)],
    )
    plsc.subcore_barrier()
    pltpu.sync_copy(dst_hbm.at[wid], idx_v)

    def grp(g, carry):
        base = g * GRP
        lds = [
            pltpu.async_copy(vals_hbm.at[wid, base + j], vals_v.at[j], lsem)
            for j in range(GRP)
        ]
        for ld in lds:
            ld.wait()
        scs = [
            pltpu.async_copy(
                vals_v.at[j], acc.at[idx_v.at[base + j]], ssem, add=True
            )
            for j in range(GRP)
        ]
        for sc in scs:
            sc.wait()
        return carry

    lax.fori_loop(0, N_CHUNKS // GRP, grp, 0)
    plsc.subcore_barrier()
    pltpu.sync_copy(
        acc.at[pl.ds(s * ACC_STRIPE, ACC_STRIPE)],
        out_hbm.at[c, pl.ds(s * ACC_STRIPE, ACC_STRIPE)],
    )


def _sc_scatter(vals4, dst3, zeros):
    mesh = plsc.VectorSubcoreMesh(core_axis_name="c", subcore_axis_name="s")
    f = pl.kernel(
        _scatter_body,
        out_type=jax.ShapeDtypeStruct((NC, N_ACC, PAD), jnp.float32),
        mesh=mesh,
        compiler_params=pltpu.CompilerParams(use_tc_tiling_on_sc=False, needs_layout_passes=False),
        scratch_types=[
            pltpu.VMEM((N_CHUNKS, CHUNK), jnp.int32),
            pltpu.VMEM((GRP, CHUNK, PAD), jnp.float32),
            pltpu.VMEM_SHARED((N_ACC, PAD), jnp.float32),
            pltpu.SemaphoreType.DMA,
            pltpu.SemaphoreType.DMA,
        ],
    )
    return f(vals4, dst3, zeros)


def _combine_body(p_ref, s_out, v_out):
    tot = p_ref[0] + p_ref[1]
    s_out[...] = tot[0:N_NODES, 9:12]
    v_out[...] = tot[0:N_NODES, 0:9]


def _combine(partials):
    return pl.pallas_call(
        _combine_body,
        in_specs=[pl.BlockSpec((NC, N_ACC, PAD), lambda: (0, 0, 0))],
        out_specs=[
            pl.BlockSpec((N_NODES, 3), lambda: (0, 0)),
            pl.BlockSpec((N_NODES, 9), lambda: (0, 0)),
        ],
        out_shape=[
            jax.ShapeDtypeStruct((N_NODES, 3), jnp.float32),
            jax.ShapeDtypeStruct((N_NODES, 9), jnp.float32),
        ],
    )(partials)


def kernel(s_j, v_j, r_ij, nbrs, W1, b1, W2, b2, Wd, bd):
    w1t = W1.T
    b1r = b1.reshape(1, FEAT)
    wp = jnp.zeros((FEAT, PAD), jnp.float32).at[:, :9].set(W2[:9].T)
    b2p = jnp.zeros((1, PAD), jnp.float32).at[0, :9].set(b2[:9])
    wd16 = jnp.zeros((PAD, N_RBF), jnp.float32).at[:9].set(Wd[:9])
    bd16 = jnp.zeros((PAD,), jnp.float32).at[:9].set(bd[:9])
    wdk = jnp.kron(wd16, jnp.eye(FOLD, dtype=jnp.float32))
    bdk = jnp.repeat(bd16, FOLD).reshape(128, 1)

    c16 = np.zeros((PAD, PAD), np.float32)
    a16 = np.zeros((PAD, PAD), np.float32)
    for i in range(3):
        for k in range(3):
            c16[3 * i + k, 6 + i] = 1.0
            a16[3 * i + k, i] = 1.0
        c16[9 + i, 3 + i] = 1.0
    ck = jnp.asarray(np.kron(c16, np.eye(8, dtype=np.float32)))
    ak = jnp.asarray(np.kron(a16, np.eye(8, dtype=np.float32)))

    pin = np.zeros((128, 128), np.float32)
    pout = np.zeros((128, 128), np.float32)
    for cc in range(8):
        for j in range(PAD):
            pin[8 * j + cc, 16 * cc + j] = 1.0
            pout[16 * cc + j, 8 * j + cc] = 1.0
    pin = jnp.asarray(pin)
    pout = jnp.asarray(pout)

    q = np.arange(N_NODES)
    qi, qt = q // NODE_BLK, q % NODE_BLK
    n_of_q = jnp.asarray((qi * NODE_RPB + (qt % NODE_RPB)) * 8 + qt // NODE_RPB)
    s_perm = jnp.take(s_j, n_of_q, axis=0)

    m = np.arange(E_PAD)
    mR, mc = m // 8, m % 8
    e_of_m = jnp.asarray((mR // FB) * EDGE_BLK + mc * FB + (mR % FB))

    npad = E_PAD - N_EDGES
    src_pad = jnp.concatenate([nbrs[:, 1], jnp.zeros((npad,), jnp.int32)])
    dst_pad = jnp.concatenate([nbrs[:, 0], jnp.full((npad,), TRASH, jnp.int32)])
    src3 = jnp.take(src_pad, e_of_m).reshape(NW, N_CHUNKS, CHUNK)
    dst3 = jnp.take(dst_pad, e_of_m).reshape(NW, N_CHUNKS, CHUNK)

    r_pad = jnp.concatenate([r_ij, jnp.zeros((npad, 3), jnp.float32)], axis=0)
    v_pad = jnp.concatenate(
        [v_j.reshape(N_EDGES, 9), jnp.zeros((npad, 9), jnp.float32)], axis=0
    )
    r_stack = (
        r_pad.T.reshape(3, N_BLOCKS, FOLD, FB).transpose(0, 2, 1, 3).reshape(24, COLS)
    )
    v_stack = (
        v_pad.T.reshape(9, N_BLOCKS, FOLD, FB).transpose(0, 2, 1, 3).reshape(72, COLS)
    )
    zeros = jnp.zeros((N_ACC, PAD), jnp.float32)

    node_tbl = _node_mlp(s_perm, w1t, b1r, wp, b2p).reshape(N_NODES, PAD)
    p4 = _sc_gather(node_tbl, src3)
    vals = _edge_stage(
        p4.reshape(E_PAD // 8, 128), r_stack, v_stack, wdk, bdk, ck, ak, pin, pout
    )
    partials = _sc_scatter(vals.reshape(NW, N_CHUNKS, CHUNK, PAD), dst3, zeros)
    ds, dv = _combine(partials)
    return (ds, dv.reshape(N_NODES, 3, 3))


# speedup vs baseline: 1.2863x; 1.0005x over previous
import functools
import math

import jax
import jax.numpy as jnp
import numpy as np
from jax import lax
from jax.experimental import pallas as pl
from jax.experimental.pallas import tpu as pltpu
from jax.experimental.pallas import tpu_sc as plsc

N_NODES = 10000
N_EDGES = 320000
FEAT = 128
N_RBF = 20
CUTOFF = 5.0

PAD = 16
E_PAD = 327680
EDGE_BLK = 8192
FOLD = 8
FB = EDGE_BLK // FOLD
N_BLOCKS = E_PAD // EDGE_BLK
COLS = E_PAD // FOLD

NC = 2
NS = 16
NW = NC * NS
E_PER_W = E_PAD // NW
CHUNK = 80
N_CHUNKS = E_PER_W // CHUNK
GRP = 4
N_ACC = 10016
TRASH = N_NODES
ACC_STRIPE = N_ACC // NS

NODE_BLK = 10000
NODE_RPB = NODE_BLK // 8


def _node_mlp_body(s_ref, w1t_ref, b1_ref, wp_ref, b2p_ref, out_ref):
    x = jnp.dot(s_ref[...], w1t_ref[...], preferred_element_type=jnp.float32)
    x = x + b1_ref[...]
    h = x * jax.nn.sigmoid(x)
    ph = jnp.dot(h, wp_ref[...], preferred_element_type=jnp.float32) + b2p_ref[...]
    out_ref[...] = jnp.concatenate(
        [ph[c * NODE_RPB : (c + 1) * NODE_RPB, :] for c in range(8)], axis=1
    )


def _node_mlp(s_perm, w1t, b1r, wp, b2p):
    return pl.pallas_call(
        _node_mlp_body,
        grid=(1,),
        in_specs=[
            pl.BlockSpec((NODE_BLK, FEAT), lambda i: (0, 0)),
            pl.BlockSpec((FEAT, FEAT), lambda i: (0, 0)),
            pl.BlockSpec((1, FEAT), lambda i: (0, 0)),
            pl.BlockSpec((FEAT, PAD), lambda i: (0, 0)),
            pl.BlockSpec((1, PAD), lambda i: (0, 0)),
        ],
        out_specs=pl.BlockSpec((NODE_RPB, 8 * PAD), lambda i: (0, 0)),
        out_shape=jax.ShapeDtypeStruct((N_NODES // 8, 8 * PAD), jnp.float32),
    )(s_perm, w1t, b1r, wp, b2p)


def _gather_body(table_hbm, idx_hbm, out_hbm, idx_v, rows_v, gsem, ssem):
    c = lax.axis_index("c")
    s = lax.axis_index("s")
    wid = s * NC + c
    pltpu.sync_copy(idx_hbm.at[wid], idx_v)

    def grp(g, carry):
        base = g * GRP
        cps = [
            pltpu.async_copy(table_hbm.at[idx_v.at[base + j]], rows_v.at[j], gsem)
            for j in range(GRP)
        ]
        for cp in cps:
            cp.wait()
        sts = [
            pltpu.async_copy(rows_v.at[j], out_hbm.at[wid, base + j], ssem)
            for j in range(GRP)
        ]
        for st in sts:
            st.wait()
        return carry

    lax.fori_loop(0, N_CHUNKS // GRP, grp, 0)


def _sc_gather(node16, src3):
    mesh = plsc.VectorSubcoreMesh(core_axis_name="c", subcore_axis_name="s")
    f = pl.kernel(
        _gather_body,
        out_type=jax.ShapeDtypeStruct((NW, N_CHUNKS, CHUNK, PAD), jnp.float32),
        mesh=mesh,
        compiler_params=pltpu.CompilerParams(use_tc_tiling_on_sc=False),
        scratch_types=[
            pltpu.VMEM((N_CHUNKS, CHUNK), jnp.int32),
            pltpu.VMEM((GRP, CHUNK, PAD), jnp.float32),
            pltpu.SemaphoreType.DMA,
            pltpu.SemaphoreType.DMA,
        ],
    )
    return f(node16, src3)


def _edge_body(p_ref, r_ref, v_ref, wdk_ref, bdk_ref, ck_ref, ak_ref,
               pin_ref, pout_ref, out_ref):
    rs = r_ref[...]
    xx = rs[0:8]
    yy = rs[8:16]
    zz = rs[16:24]
    d2 = xx * xx + yy * yy + zz * zz
    d = jnp.sqrt(d2)
    d3 = jnp.concatenate([d, d, d], axis=0)
    d23 = jnp.concatenate([d2, d2, d2], axis=0)
    unit = jnp.where(d23 == 0.0, 0.0, rs / d3)
    idm = jnp.where(d2 == 0.0, 0.0, 1.0 / d)
    x1 = d * (math.pi / CUTOFF)
    s1 = jnp.sin(x1)
    c1 = jnp.cos(x1)
    two_c1 = c1 + c1
    terms = [s1, two_c1 * s1]
    for _ in range(N_RBF - 2):
        terms.append(two_c1 * terms[-1] - terms[-2])
    rbf = jnp.concatenate([t * idm for t in terms], axis=0)
    w = jnp.dot(wdk_ref[...], rbf, preferred_element_type=jnp.float32) + bdk_ref[...]
    p_stack = jnp.dot(
        pin_ref[...], p_ref[...].T, preferred_element_type=jnp.float32
    )
    P = p_stack * w
    u = jnp.concatenate(
        [unit[8 * k : 8 * (k + 1)] for _i in range(3) for k in range(3)]
        + [jnp.ones((24, FB), jnp.float32), jnp.zeros((32, FB), jnp.float32)],
        axis=0,
    )
    vpad = jnp.concatenate(
        [v_ref[...], jnp.zeros((128 - 72, FB), jnp.float32)], axis=0
    )
    outt = (
        jnp.dot(ck_ref[...], P, preferred_element_type=jnp.float32) * u
        + jnp.dot(ak_ref[...], P, preferred_element_type=jnp.float32) * vpad
    )
    out_ref[...] = jnp.dot(
        pout_ref[...], outt, preferred_element_type=jnp.float32
    ).T


def _edge_stage(p, r_stack, v_stack, wdk, bdk, ck, ak, pin, pout):
    return pl.pallas_call(
        _edge_body,
        grid=(N_BLOCKS,),
        in_specs=[
            pl.BlockSpec((FB, 128), lambda i: (i, 0)),
            pl.BlockSpec((24, FB), lambda i: (0, i)),
            pl.BlockSpec((72, FB), lambda i: (0, i)),
            pl.BlockSpec((128, 8 * N_RBF), lambda i: (0, 0)),
            pl.BlockSpec((128, 1), lambda i: (0, 0)),
            pl.BlockSpec((128, 128), lambda i: (0, 0)),
            pl.BlockSpec((128, 128), lambda i: (0, 0)),
            pl.BlockSpec((128, 128), lambda i: (0, 0)),
            pl.BlockSpec((128, 128), lambda i: (0, 0)),
        ],
        out_specs=pl.BlockSpec((FB, 128), lambda i: (i, 0)),
        out_shape=jax.ShapeDtypeStruct((E_PAD // 8, 128), jnp.float32),
    )(p, r_stack, v_stack, wdk, bdk, ck, ak, pin, pout)


def _scatter_body(vals_hbm, dst_hbm, zeros_hbm, out_hbm, idx_v, vals_v,
                  acc, lsem, ssem):
    c = lax.axis_index("c")
    s = lax.axis_index("s")
    wid = s * NC + c
    pltpu.sync_copy(
        zeros_hbm.at[pl.ds(s * ACC_STRIPE, ACC_STRIPE)],
        acc.at[pl.ds(s * ACC_STRIPE, ACC_STRIPE)],
    )
    plsc.subcore_barrier()
    pltpu.sync_copy(dst_hbm.at[wid], idx_v)

    def grp(g, carry):
        base = g * GRP
        lds = [
            pltpu.async_copy(vals_hbm.at[wid, base + j], vals_v.at[j], lsem)
            for j in range(GRP)
        ]
        for ld in lds:
            ld.wait()
        scs = [
            pltpu.async_copy(
                vals_v.at[j], acc.at[idx_v.at[base + j]], ssem, add=True
            )
            for j in range(GRP)
        ]
        for sc in scs:
            sc.wait()
        return carry

    lax.fori_loop(0, N_CHUNKS // GRP, grp, 0)
    plsc.subcore_barrier()
    pltpu.sync_copy(
        acc.at[pl.ds(s * ACC_STRIPE, ACC_STRIPE)],
        out_hbm.at[c, pl.ds(s * ACC_STRIPE, ACC_STRIPE)],
    )


def _sc_scatter(vals4, dst3, zeros):
    mesh = plsc.VectorSubcoreMesh(core_axis_name="c", subcore_axis_name="s")
    f = pl.kernel(
        _scatter_body,
        out_type=jax.ShapeDtypeStruct((NC, N_ACC, PAD), jnp.float32),
        mesh=mesh,
        compiler_params=pltpu.CompilerParams(use_tc_tiling_on_sc=False),
        scratch_types=[
            pltpu.VMEM((N_CHUNKS, CHUNK), jnp.int32),
            pltpu.VMEM((GRP, CHUNK, PAD), jnp.float32),
            pltpu.VMEM_SHARED((N_ACC, PAD), jnp.float32),
            pltpu.SemaphoreType.DMA,
            pltpu.SemaphoreType.DMA,
        ],
    )
    return f(vals4, dst3, zeros)


def _combine_body(p_ref, s_out, v_out):
    tot = p_ref[0] + p_ref[1]
    s_out[...] = tot[0:N_NODES, 9:12]
    v_out[...] = tot[0:N_NODES, 0:9]


def _combine(partials):
    return pl.pallas_call(
        _combine_body,
        in_specs=[pl.BlockSpec((NC, N_ACC, PAD), lambda: (0, 0, 0))],
        out_specs=[
            pl.BlockSpec((N_NODES, 3), lambda: (0, 0)),
            pl.BlockSpec((N_NODES, 9), lambda: (0, 0)),
        ],
        out_shape=[
            jax.ShapeDtypeStruct((N_NODES, 3), jnp.float32),
            jax.ShapeDtypeStruct((N_NODES, 9), jnp.float32),
        ],
    )(partials)


def kernel(s_j, v_j, r_ij, nbrs, W1, b1, W2, b2, Wd, bd):
    w1t = W1.T
    b1r = b1.reshape(1, FEAT)
    wp = jnp.zeros((FEAT, PAD), jnp.float32).at[:, :9].set(W2[:9].T)
    b2p = jnp.zeros((1, PAD), jnp.float32).at[0, :9].set(b2[:9])
    wd16 = jnp.zeros((PAD, N_RBF), jnp.float32).at[:9].set(Wd[:9])
    bd16 = jnp.zeros((PAD,), jnp.float32).at[:9].set(bd[:9])
    wdk = jnp.kron(wd16, jnp.eye(FOLD, dtype=jnp.float32))
    bdk = jnp.repeat(bd16, FOLD).reshape(128, 1)

    c16 = np.zeros((PAD, PAD), np.float32)
    a16 = np.zeros((PAD, PAD), np.float32)
    for i in range(3):
        for k in range(3):
            c16[3 * i + k, 6 + i] = 1.0
            a16[3 * i + k, i] = 1.0
        c16[9 + i, 3 + i] = 1.0
    ck = jnp.asarray(np.kron(c16, np.eye(8, dtype=np.float32)))
    ak = jnp.asarray(np.kron(a16, np.eye(8, dtype=np.float32)))

    pin = np.zeros((128, 128), np.float32)
    pout = np.zeros((128, 128), np.float32)
    for cc in range(8):
        for j in range(PAD):
            pin[8 * j + cc, 16 * cc + j] = 1.0
            pout[16 * cc + j, 8 * j + cc] = 1.0
    pin = jnp.asarray(pin)
    pout = jnp.asarray(pout)

    q = np.arange(N_NODES)
    qi, qt = q // NODE_BLK, q % NODE_BLK
    n_of_q = jnp.asarray((qi * NODE_RPB + (qt % NODE_RPB)) * 8 + qt // NODE_RPB)
    s_perm = jnp.take(s_j, n_of_q, axis=0)

    m = np.arange(E_PAD)
    mR, mc = m // 8, m % 8
    e_of_m = jnp.asarray((mR // FB) * EDGE_BLK + mc * FB + (mR % FB))

    npad = E_PAD - N_EDGES
    src_pad = jnp.concatenate([nbrs[:, 1], jnp.zeros((npad,), jnp.int32)])
    dst_pad = jnp.concatenate([nbrs[:, 0], jnp.full((npad,), TRASH, jnp.int32)])
    src3 = jnp.take(src_pad, e_of_m).reshape(NW, N_CHUNKS, CHUNK)
    dst3 = jnp.take(dst_pad, e_of_m).reshape(NW, N_CHUNKS, CHUNK)

    r_pad = jnp.concatenate([r_ij, jnp.zeros((npad, 3), jnp.float32)], axis=0)
    v_pad = jnp.concatenate(
        [v_j.reshape(N_EDGES, 9), jnp.zeros((npad, 9), jnp.float32)], axis=0
    )
    r_stack = (
        r_pad.T.reshape(3, N_BLOCKS, FOLD, FB).transpose(0, 2, 1, 3).reshape(24, COLS)
    )
    v_stack = (
        v_pad.T.reshape(9, N_BLOCKS, FOLD, FB).transpose(0, 2, 1, 3).reshape(72, COLS)
    )
    zeros = jnp.zeros((N_ACC, PAD), jnp.float32)

    node_tbl = _node_mlp(s_perm, w1t, b1r, wp, b2p).reshape(N_NODES, PAD)
    p4 = _sc_gather(node_tbl, src3)
    vals = _edge_stage(
        p4.reshape(E_PAD // 8, 128), r_stack, v_stack, wdk, bdk, ck, ak, pin, pout
    )
    partials = _sc_scatter(vals.reshape(NW, N_CHUNKS, CHUNK, PAD), dst3, zeros)
    ds, dv = _combine(partials)
    return (ds, dv.reshape(N_NODES, 3, 3))


# speedup vs baseline: 1.3475x; 1.0476x over previous
import functools
import math

import jax
import jax.numpy as jnp
import numpy as np
from jax import lax
from jax.experimental import pallas as pl
from jax.experimental.pallas import tpu as pltpu
from jax.experimental.pallas import tpu_sc as plsc

N_NODES = 10000
N_EDGES = 320000
FEAT = 128
N_RBF = 20
CUTOFF = 5.0

PAD = 16
E_PAD = 327680
EDGE_BLK = 8192
FOLD = 8
FB = EDGE_BLK // FOLD
N_BLOCKS = E_PAD // EDGE_BLK
COLS = E_PAD // FOLD

NC = 2
NS = 16
NW = NC * NS
E_PER_W = E_PAD // NW
CHUNK = 80
N_CHUNKS = E_PER_W // CHUNK
GRP = 8
N_ACC = 10016
TRASH = N_NODES
ACC_STRIPE = N_ACC // NS

NODE_BLK = 10000
NODE_RPB = NODE_BLK // 8


def _node_mlp_body(s_ref, w1t_ref, b1_ref, wp_ref, b2p_ref, out_ref):
    x = jnp.dot(s_ref[...], w1t_ref[...], preferred_element_type=jnp.float32)
    x = x + b1_ref[...]
    h = x * jax.nn.sigmoid(x)
    ph = jnp.dot(h, wp_ref[...], preferred_element_type=jnp.float32) + b2p_ref[...]
    out_ref[...] = jnp.concatenate(
        [ph[c * NODE_RPB : (c + 1) * NODE_RPB, :] for c in range(8)], axis=1
    )


def _node_mlp(s_perm, w1t, b1r, wp, b2p):
    return pl.pallas_call(
        _node_mlp_body,
        grid=(1,),
        in_specs=[
            pl.BlockSpec((NODE_BLK, FEAT), lambda i: (0, 0)),
            pl.BlockSpec((FEAT, FEAT), lambda i: (0, 0)),
            pl.BlockSpec((1, FEAT), lambda i: (0, 0)),
            pl.BlockSpec((FEAT, PAD), lambda i: (0, 0)),
            pl.BlockSpec((1, PAD), lambda i: (0, 0)),
        ],
        out_specs=pl.BlockSpec((NODE_RPB, 8 * PAD), lambda i: (0, 0)),
        out_shape=jax.ShapeDtypeStruct((N_NODES // 8, 8 * PAD), jnp.float32),
    )(s_perm, w1t, b1r, wp, b2p)


def _gather_body(table_hbm, idx_hbm, out_hbm, idx_v, rows_v, gsem, ssem):
    c = lax.axis_index("c")
    s = lax.axis_index("s")
    wid = s * NC + c
    pltpu.sync_copy(idx_hbm.at[wid], idx_v)

    def grp(g, carry):
        base = g * GRP
        cps = [
            pltpu.async_copy(table_hbm.at[idx_v.at[base + j]], rows_v.at[j], gsem)
            for j in range(GRP)
        ]
        for cp in cps:
            cp.wait()
        sts = [
            pltpu.async_copy(rows_v.at[j], out_hbm.at[wid, base + j], ssem)
            for j in range(GRP)
        ]
        for st in sts:
            st.wait()
        return carry

    lax.fori_loop(0, N_CHUNKS // GRP, grp, 0)


def _sc_gather(node16, src3):
    mesh = plsc.VectorSubcoreMesh(core_axis_name="c", subcore_axis_name="s")
    f = pl.kernel(
        _gather_body,
        out_type=jax.ShapeDtypeStruct((NW, N_CHUNKS, CHUNK, PAD), jnp.float32),
        mesh=mesh,
        compiler_params=pltpu.CompilerParams(use_tc_tiling_on_sc=False),
        scratch_types=[
            pltpu.VMEM((N_CHUNKS, CHUNK), jnp.int32),
            pltpu.VMEM((GRP, CHUNK, PAD), jnp.float32),
            pltpu.SemaphoreType.DMA,
            pltpu.SemaphoreType.DMA,
        ],
    )
    return f(node16, src3)


def _edge_body(p_ref, r_ref, v_ref, wdk_ref, bdk_ref, ck_ref, ak_ref,
               pin_ref, pout_ref, out_ref):
    rs = r_ref[...]
    xx = rs[0:8]
    yy = rs[8:16]
    zz = rs[16:24]
    d2 = xx * xx + yy * yy + zz * zz
    d = jnp.sqrt(d2)
    d3 = jnp.concatenate([d, d, d], axis=0)
    d23 = jnp.concatenate([d2, d2, d2], axis=0)
    unit = jnp.where(d23 == 0.0, 0.0, rs / d3)
    idm = jnp.where(d2 == 0.0, 0.0, 1.0 / d)
    x1 = d * (math.pi / CUTOFF)
    s1 = jnp.sin(x1)
    c1 = jnp.cos(x1)
    two_c1 = c1 + c1
    terms = [s1, two_c1 * s1]
    for _ in range(N_RBF - 2):
        terms.append(two_c1 * terms[-1] - terms[-2])
    rbf = jnp.concatenate([t * idm for t in terms], axis=0)
    w = jnp.dot(wdk_ref[...], rbf, preferred_element_type=jnp.float32) + bdk_ref[...]
    p_stack = jnp.dot(
        pin_ref[...], p_ref[...].T, preferred_element_type=jnp.float32
    )
    P = p_stack * w
    u = jnp.concatenate(
        [unit[8 * k : 8 * (k + 1)] for _i in range(3) for k in range(3)]
        + [jnp.ones((24, FB), jnp.float32), jnp.zeros((32, FB), jnp.float32)],
        axis=0,
    )
    vpad = jnp.concatenate(
        [v_ref[...], jnp.zeros((128 - 72, FB), jnp.float32)], axis=0
    )
    outt = (
        jnp.dot(ck_ref[...], P, preferred_element_type=jnp.float32) * u
        + jnp.dot(ak_ref[...], P, preferred_element_type=jnp.float32) * vpad
    )
    out_ref[...] = jnp.dot(
        pout_ref[...], outt, preferred_element_type=jnp.float32
    ).T


def _edge_stage(p, r_stack, v_stack, wdk, bdk, ck, ak, pin, pout):
    return pl.pallas_call(
        _edge_body,
        grid=(N_BLOCKS,),
        in_specs=[
            pl.BlockSpec((FB, 128), lambda i: (i, 0)),
            pl.BlockSpec((24, FB), lambda i: (0, i)),
            pl.BlockSpec((72, FB), lambda i: (0, i)),
            pl.BlockSpec((128, 8 * N_RBF), lambda i: (0, 0)),
            pl.BlockSpec((128, 1), lambda i: (0, 0)),
            pl.BlockSpec((128, 128), lambda i: (0, 0)),
            pl.BlockSpec((128, 128), lambda i: (0, 0)),
            pl.BlockSpec((128, 128), lambda i: (0, 0)),
            pl.BlockSpec((128, 128), lambda i: (0, 0)),
        ],
        out_specs=pl.BlockSpec((FB, 128), lambda i: (i, 0)),
        out_shape=jax.ShapeDtypeStruct((E_PAD // 8, 128), jnp.float32),
    )(p, r_stack, v_stack, wdk, bdk, ck, ak, pin, pout)


def _scatter_body(vals_hbm, dst_hbm, zeros_hbm, out_hbm, idx_v, vals_v,
                  acc, lsem, ssem):
    c = lax.axis_index("c")
    s = lax.axis_index("s")
    wid = s * NC + c
    pltpu.sync_copy(
        zeros_hbm.at[pl.ds(s * ACC_STRIPE, ACC_STRIPE)],
        acc.at[pl.ds(s * ACC_STRIPE, ACC_STRIPE)],
    )
    plsc.subcore_barrier()
    pltpu.sync_copy(dst_hbm.at[wid], idx_v)

    def grp(g, carry):
        base = g * GRP
        lds = [
            pltpu.async_copy(vals_hbm.at[wid, base + j], vals_v.at[j], lsem)
            for j in range(GRP)
        ]
        for ld in lds:
            ld.wait()
        scs = [
            pltpu.async_copy(
                vals_v.at[j], acc.at[idx_v.at[base + j]], ssem, add=True
            )
            for j in range(GRP)
        ]
        for sc in scs:
            sc.wait()
        return carry

    lax.fori_loop(0, N_CHUNKS // GRP, grp, 0)
    plsc.subcore_barrier()
    pltpu.sync_copy(
        acc.at[pl.ds(s * ACC_STRIPE, ACC_STRIPE)],
        out_hbm.at[c, pl.ds(s * ACC_STRIPE, ACC_STRIPE)],
    )


def _sc_scatter(vals4, dst3, zeros):
    mesh = plsc.VectorSubcoreMesh(core_axis_name="c", subcore_axis_name="s")
    f = pl.kernel(
        _scatter_body,
        out_type=jax.ShapeDtypeStruct((NC, N_ACC, PAD), jnp.float32),
        mesh=mesh,
        compiler_params=pltpu.CompilerParams(use_tc_tiling_on_sc=False),
        scratch_types=[
            pltpu.VMEM((N_CHUNKS, CHUNK), jnp.int32),
            pltpu.VMEM((GRP, CHUNK, PAD), jnp.float32),
            pltpu.VMEM_SHARED((N_ACC, PAD), jnp.float32),
            pltpu.SemaphoreType.DMA,
            pltpu.SemaphoreType.DMA,
        ],
    )
    return f(vals4, dst3, zeros)


def _combine_body(p_ref, s_out, v_out):
    tot = p_ref[0] + p_ref[1]
    s_out[...] = tot[0:N_NODES, 9:12]
    v_out[...] = tot[0:N_NODES, 0:9]


def _combine(partials):
    return pl.pallas_call(
        _combine_body,
        in_specs=[pl.BlockSpec((NC, N_ACC, PAD), lambda: (0, 0, 0))],
        out_specs=[
            pl.BlockSpec((N_NODES, 3), lambda: (0, 0)),
            pl.BlockSpec((N_NODES, 9), lambda: (0, 0)),
        ],
        out_shape=[
            jax.ShapeDtypeStruct((N_NODES, 3), jnp.float32),
            jax.ShapeDtypeStruct((N_NODES, 9), jnp.float32),
        ],
    )(partials)


def kernel(s_j, v_j, r_ij, nbrs, W1, b1, W2, b2, Wd, bd):
    w1t = W1.T
    b1r = b1.reshape(1, FEAT)
    wp = jnp.zeros((FEAT, PAD), jnp.float32).at[:, :9].set(W2[:9].T)
    b2p = jnp.zeros((1, PAD), jnp.float32).at[0, :9].set(b2[:9])
    wd16 = jnp.zeros((PAD, N_RBF), jnp.float32).at[:9].set(Wd[:9])
    bd16 = jnp.zeros((PAD,), jnp.float32).at[:9].set(bd[:9])
    wdk = jnp.kron(wd16, jnp.eye(FOLD, dtype=jnp.float32))
    bdk = jnp.repeat(bd16, FOLD).reshape(128, 1)

    c16 = np.zeros((PAD, PAD), np.float32)
    a16 = np.zeros((PAD, PAD), np.float32)
    for i in range(3):
        for k in range(3):
            c16[3 * i + k, 6 + i] = 1.0
            a16[3 * i + k, i] = 1.0
        c16[9 + i, 3 + i] = 1.0
    ck = jnp.asarray(np.kron(c16, np.eye(8, dtype=np.float32)))
    ak = jnp.asarray(np.kron(a16, np.eye(8, dtype=np.float32)))

    pin = np.zeros((128, 128), np.float32)
    pout = np.zeros((128, 128), np.float32)
    for cc in range(8):
        for j in range(PAD):
            pin[8 * j + cc, 16 * cc + j] = 1.0
            pout[16 * cc + j, 8 * j + cc] = 1.0
    pin = jnp.asarray(pin)
    pout = jnp.asarray(pout)

    q = np.arange(N_NODES)
    qi, qt = q // NODE_BLK, q % NODE_BLK
    n_of_q = jnp.asarray((qi * NODE_RPB + (qt % NODE_RPB)) * 8 + qt // NODE_RPB)
    s_perm = jnp.take(s_j, n_of_q, axis=0)

    m = np.arange(E_PAD)
    mR, mc = m // 8, m % 8
    e_of_m = jnp.asarray((mR // FB) * EDGE_BLK + mc * FB + (mR % FB))

    npad = E_PAD - N_EDGES
    src_pad = jnp.concatenate([nbrs[:, 1], jnp.zeros((npad,), jnp.int32)])
    dst_pad = jnp.concatenate([nbrs[:, 0], jnp.full((npad,), TRASH, jnp.int32)])
    src3 = jnp.take(src_pad, e_of_m).reshape(NW, N_CHUNKS, CHUNK)
    dst3 = jnp.take(dst_pad, e_of_m).reshape(NW, N_CHUNKS, CHUNK)

    r_pad = jnp.concatenate([r_ij, jnp.zeros((npad, 3), jnp.float32)], axis=0)
    v_pad = jnp.concatenate(
        [v_j.reshape(N_EDGES, 9), jnp.zeros((npad, 9), jnp.float32)], axis=0
    )
    r_stack = (
        r_pad.T.reshape(3, N_BLOCKS, FOLD, FB).transpose(0, 2, 1, 3).reshape(24, COLS)
    )
    v_stack = (
        v_pad.T.reshape(9, N_BLOCKS, FOLD, FB).transpose(0, 2, 1, 3).reshape(72, COLS)
    )
    zeros = jnp.zeros((N_ACC, PAD), jnp.float32)

    node_tbl = _node_mlp(s_perm, w1t, b1r, wp, b2p).reshape(N_NODES, PAD)
    p4 = _sc_gather(node_tbl, src3)
    vals = _edge_stage(
        p4.reshape(E_PAD // 8, 128), r_stack, v_stack, wdk, bdk, ck, ak, pin, pout
    )
    partials = _sc_scatter(vals.reshape(NW, N_CHUNKS, CHUNK, PAD), dst3, zeros)
    ds, dv = _combine(partials)
    return (ds, dv.reshape(N_NODES, 3, 3))


# speedup vs baseline: 1.3501x; 1.0019x over previous
import math

import jax
import jax.numpy as jnp
import numpy as np
from jax import lax
from jax.experimental import pallas as pl
from jax.experimental.pallas import tpu as pltpu
from jax.experimental.pallas import tpu_sc as plsc

N_NODES = 10000
N_EDGES = 320000
FEAT = 128
N_RBF = 20
CUTOFF = 5.0

PAD = 16
E_PAD = 327680
EDGE_BLK = 8192
FOLD = 8
FB = EDGE_BLK // FOLD
N_BLOCKS = E_PAD // EDGE_BLK
COLS = E_PAD // FOLD

NC = 2
NS = 16
NW = NC * NS
E_PER_W = E_PAD // NW
CHUNK = 80
N_CHUNKS = E_PER_W // CHUNK
GRP = 8
N_ACC = 10016
TRASH = N_NODES
ACC_STRIPE = N_ACC // NS

NODE_BLK = 10000
NODE_RPB = NODE_BLK // 8


def _node_mlp_body(s_ref, w1t_ref, b1_ref, wp_ref, b2p_ref, out_ref):
    x = jnp.dot(s_ref[...], w1t_ref[...], preferred_element_type=jnp.float32)
    x = x + b1_ref[...]
    h = x * jax.nn.sigmoid(x)
    ph = jnp.dot(h, wp_ref[...], preferred_element_type=jnp.float32) + b2p_ref[...]
    out_ref[...] = jnp.concatenate(
        [ph[c * NODE_RPB : (c + 1) * NODE_RPB, :] for c in range(8)], axis=1
    )


def _node_mlp(s_perm, w1t, b1r, wp, b2p):
    return pl.pallas_call(
        _node_mlp_body,
        grid=(1,),
        in_specs=[
            pl.BlockSpec((NODE_BLK, FEAT), lambda i: (0, 0)),
            pl.BlockSpec((FEAT, FEAT), lambda i: (0, 0)),
            pl.BlockSpec((1, FEAT), lambda i: (0, 0)),
            pl.BlockSpec((FEAT, PAD), lambda i: (0, 0)),
            pl.BlockSpec((1, PAD), lambda i: (0, 0)),
        ],
        out_specs=pl.BlockSpec((NODE_RPB, 8 * PAD), lambda i: (0, 0)),
        out_shape=jax.ShapeDtypeStruct((N_NODES // 8, 8 * PAD), jnp.float32),
    )(s_perm, w1t, b1r, wp, b2p)


def _gather_body(table_hbm, idx_hbm, out_hbm, idx_v, rows_v, gsem, ssem):
    c = lax.axis_index("c")
    s = lax.axis_index("s")
    wid = s * NC + c
    pltpu.sync_copy(idx_hbm.at[wid], idx_v)

    def grp(g, carry):
        base = g * GRP
        cps = [
            pltpu.async_copy(table_hbm.at[idx_v.at[base + j]], rows_v.at[j], gsem)
            for j in range(GRP)
        ]
        for cp in cps:
            cp.wait()
        sts = [
            pltpu.async_copy(rows_v.at[j], out_hbm.at[wid, base + j], ssem)
            for j in range(GRP)
        ]
        for st in sts:
            st.wait()
        return carry

    lax.fori_loop(0, N_CHUNKS // GRP, grp, 0)


def _sc_gather(node16, src3):
    mesh = plsc.VectorSubcoreMesh(core_axis_name="c", subcore_axis_name="s")
    f = pl.kernel(
        _gather_body,
        out_type=jax.ShapeDtypeStruct((NW, N_CHUNKS, CHUNK, PAD), jnp.float32),
        mesh=mesh,
        compiler_params=pltpu.CompilerParams(use_tc_tiling_on_sc=False),
        scratch_types=[
            pltpu.VMEM((N_CHUNKS, CHUNK), jnp.int32),
            pltpu.VMEM((GRP, CHUNK, PAD), jnp.float32),
            pltpu.SemaphoreType.DMA,
            pltpu.SemaphoreType.DMA,
        ],
    )
    return f(node16, src3)


def _edge_body(p_ref, r_ref, v_ref, wdk_ref, bdk_ref, ck_ref, ak_ref,
               pin_ref, pout_ref, out_ref):
    rs = r_ref[...]
    xx = rs[0:8]
    yy = rs[8:16]
    zz = rs[16:24]
    d2 = xx * xx + yy * yy + zz * zz
    d = jnp.sqrt(d2)
    d3 = jnp.concatenate([d, d, d], axis=0)
    d23 = jnp.concatenate([d2, d2, d2], axis=0)
    unit = jnp.where(d23 == 0.0, 0.0, rs / d3)
    idm = jnp.where(d2 == 0.0, 0.0, 1.0 / d)
    x1 = d * (math.pi / CUTOFF)
    s1 = jnp.sin(x1)
    c1 = jnp.cos(x1)
    two_c1 = c1 + c1
    terms = [s1, two_c1 * s1]
    for _ in range(N_RBF - 2):
        terms.append(two_c1 * terms[-1] - terms[-2])
    rbf = jnp.concatenate([t * idm for t in terms], axis=0)
    w = jnp.dot(wdk_ref[...], rbf, preferred_element_type=jnp.float32) + bdk_ref[...]
    p_stack = jnp.dot(
        pin_ref[...], p_ref[...].T, preferred_element_type=jnp.float32
    )
    P = p_stack * w
    u = jnp.concatenate(
        [unit[8 * k : 8 * (k + 1)] for _i in range(3) for k in range(3)]
        + [jnp.ones((24, FB), jnp.float32), jnp.zeros((32, FB), jnp.float32)],
        axis=0,
    )
    vpad = jnp.concatenate(
        [v_ref[...], jnp.zeros((128 - 72, FB), jnp.float32)], axis=0
    )
    outt = (
        jnp.dot(ck_ref[...], P, preferred_element_type=jnp.float32) * u
        + jnp.dot(ak_ref[...], P, preferred_element_type=jnp.float32) * vpad
    )
    out_ref[...] = jnp.dot(
        pout_ref[...], outt, preferred_element_type=jnp.float32
    ).T


def _edge_stage(p, r_stack, v_stack, wdk, bdk, ck, ak, pin, pout):
    return pl.pallas_call(
        _edge_body,
        grid=(N_BLOCKS,),
        in_specs=[
            pl.BlockSpec((FB, 128), lambda i: (i, 0)),
            pl.BlockSpec((24, FB), lambda i: (0, i)),
            pl.BlockSpec((72, FB), lambda i: (0, i)),
            pl.BlockSpec((128, 8 * N_RBF), lambda i: (0, 0)),
            pl.BlockSpec((128, 1), lambda i: (0, 0)),
            pl.BlockSpec((128, 128), lambda i: (0, 0)),
            pl.BlockSpec((128, 128), lambda i: (0, 0)),
            pl.BlockSpec((128, 128), lambda i: (0, 0)),
            pl.BlockSpec((128, 128), lambda i: (0, 0)),
        ],
        out_specs=pl.BlockSpec((FB, 128), lambda i: (i, 0)),
        out_shape=jax.ShapeDtypeStruct((E_PAD // 8, 128), jnp.float32),
    )(p, r_stack, v_stack, wdk, bdk, ck, ak, pin, pout)


def _scatter_body(vals_hbm, dst_hbm, zeros_hbm, out_hbm, idx_v, vals_v,
                  acc, lsem, ssem):
    c = lax.axis_index("c")
    s = lax.axis_index("s")
    wid = s * NC + c
    pltpu.sync_copy(
        zeros_hbm.at[pl.ds(s * ACC_STRIPE, ACC_STRIPE)],
        acc.at[pl.ds(s * ACC_STRIPE, ACC_STRIPE)],
    )
    plsc.subcore_barrier()
    pltpu.sync_copy(dst_hbm.at[wid], idx_v)

    def grp(g, carry):
        base = g * GRP
        lds = [
            pltpu.async_copy(vals_hbm.at[wid, base + j], vals_v.at[j], lsem)
            for j in range(GRP)
        ]
        for ld in lds:
            ld.wait()
        scs = [
            pltpu.async_copy(
                vals_v.at[j], acc.at[idx_v.at[base + j]], ssem, add=True
            )
            for j in range(GRP)
        ]
        for sc in scs:
            sc.wait()
        return carry

    lax.fori_loop(0, N_CHUNKS // GRP, grp, 0)
    plsc.subcore_barrier()
    pltpu.sync_copy(
        acc.at[pl.ds(s * ACC_STRIPE, ACC_STRIPE)],
        out_hbm.at[c, pl.ds(s * ACC_STRIPE, ACC_STRIPE)],
    )


def _sc_scatter(vals4, dst3, zeros):
    mesh = plsc.VectorSubcoreMesh(core_axis_name="c", subcore_axis_name="s")
    f = pl.kernel(
        _scatter_body,
        out_type=jax.ShapeDtypeStruct((NC, N_ACC, PAD), jnp.float32),
        mesh=mesh,
        compiler_params=pltpu.CompilerParams(use_tc_tiling_on_sc=False),
        scratch_types=[
            pltpu.VMEM((N_CHUNKS, CHUNK), jnp.int32),
            pltpu.VMEM((GRP, CHUNK, PAD), jnp.float32),
            pltpu.VMEM_SHARED((N_ACC, PAD), jnp.float32),
            pltpu.SemaphoreType.DMA,
            pltpu.SemaphoreType.DMA,
        ],
    )
    return f(vals4, dst3, zeros)


def _combine_body(p_ref, s_out, v_out):
    tot = p_ref[0] + p_ref[1]
    s_out[...] = tot[0:N_NODES, 9:12]
    v_out[...] = tot[0:N_NODES, 0:9]


def _combine(partials):
    return pl.pallas_call(
        _combine_body,
        in_specs=[pl.BlockSpec((NC, N_ACC, PAD), lambda: (0, 0, 0))],
        out_specs=[
            pl.BlockSpec((N_NODES, 3), lambda: (0, 0)),
            pl.BlockSpec((N_NODES, 9), lambda: (0, 0)),
        ],
        out_shape=[
            jax.ShapeDtypeStruct((N_NODES, 3), jnp.float32),
            jax.ShapeDtypeStruct((N_NODES, 9), jnp.float32),
        ],
    )(partials)


def kernel(s_j, v_j, r_ij, nbrs, W1, b1, W2, b2, Wd, bd):
    w1t = W1.T
    b1r = b1.reshape(1, FEAT)
    wp = jnp.zeros((FEAT, PAD), jnp.float32).at[:, :9].set(W2[:9].T)
    b2p = jnp.zeros((1, PAD), jnp.float32).at[0, :9].set(b2[:9])
    wd16 = jnp.zeros((PAD, N_RBF), jnp.float32).at[:9].set(Wd[:9])
    bd16 = jnp.zeros((PAD,), jnp.float32).at[:9].set(bd[:9])
    wdk = jnp.kron(wd16, jnp.eye(FOLD, dtype=jnp.float32))
    bdk = jnp.repeat(bd16, FOLD).reshape(128, 1)

    c16 = np.zeros((PAD, PAD), np.float32)
    a16 = np.zeros((PAD, PAD), np.float32)
    for i in range(3):
        for k in range(3):
            c16[3 * i + k, 6 + i] = 1.0
            a16[3 * i + k, i] = 1.0
        c16[9 + i, 3 + i] = 1.0
    ck = jnp.asarray(np.kron(c16, np.eye(8, dtype=np.float32)))
    ak = jnp.asarray(np.kron(a16, np.eye(8, dtype=np.float32)))

    pin = np.zeros((128, 128), np.float32)
    pout = np.zeros((128, 128), np.float32)
    for cc in range(8):
        for j in range(PAD):
            pin[8 * j + cc, 16 * cc + j] = 1.0
            pout[16 * cc + j, 8 * j + cc] = 1.0
    pin = jnp.asarray(pin)
    pout = jnp.asarray(pout)

    q = np.arange(N_NODES)
    qi, qt = q // NODE_BLK, q % NODE_BLK
    n_of_q = jnp.asarray((qi * NODE_RPB + (qt % NODE_RPB)) * 8 + qt // NODE_RPB)
    s_perm = jnp.take(s_j, n_of_q, axis=0)

    m = np.arange(E_PAD)
    mR, mc = m // 8, m % 8
    e_of_m = jnp.asarray((mR // FB) * EDGE_BLK + mc * FB + (mR % FB))

    npad = E_PAD - N_EDGES
    src_pad = jnp.concatenate([nbrs[:, 1], jnp.zeros((npad,), jnp.int32)])
    dst_pad = jnp.concatenate([nbrs[:, 0], jnp.full((npad,), TRASH, jnp.int32)])
    src3 = jnp.take(src_pad, e_of_m).reshape(NW, N_CHUNKS, CHUNK)
    dst3 = jnp.take(dst_pad, e_of_m).reshape(NW, N_CHUNKS, CHUNK)

    r_pad = jnp.concatenate([r_ij, jnp.zeros((npad, 3), jnp.float32)], axis=0)
    v_pad = jnp.concatenate(
        [v_j.reshape(N_EDGES, 9), jnp.zeros((npad, 9), jnp.float32)], axis=0
    )
    r_stack = (
        r_pad.T.reshape(3, N_BLOCKS, FOLD, FB).transpose(0, 2, 1, 3).reshape(24, COLS)
    )
    v_stack = (
        v_pad.T.reshape(9, N_BLOCKS, FOLD, FB).transpose(0, 2, 1, 3).reshape(72, COLS)
    )
    zeros = jnp.zeros((N_ACC, PAD), jnp.float32)

    node_tbl = _node_mlp(s_perm, w1t, b1r, wp, b2p).reshape(N_NODES, PAD)
    p4 = _sc_gather(node_tbl, src3)
    vals = _edge_stage(
        p4.reshape(E_PAD // 8, 128), r_stack, v_stack, wdk, bdk, ck, ak, pin, pout
    )
    partials = _sc_scatter(vals.reshape(NW, N_CHUNKS, CHUNK, PAD), dst3, zeros)
    ds, dv = _combine(partials)
    return (ds, dv.reshape(N_NODES, 3, 3))
